# Initial kernel scaffold; baseline (speedup 1.0000x reference)
import functools

import jax
import jax.numpy as jnp
from jax import lax
from jax.experimental import pallas as pl
from jax.experimental.pallas import tpu as pltpu
from jax.experimental.pallas import tpu_sc as plsc

N = 10000
E = 320000
D = 128
NC = 2
NS = 16
NTILES = NC * NS
NP = 10240
RH = 160
NPASS = 2
ET1 = E // NTILES
CK1 = 2000
C2 = 2560
NCH2 = E // C2
B = 128
BUFCAP = 2816
STRIP = NP // NTILES
EPS = 1e-6

_mesh = plsc.VectorSubcoreMesh(core_axis_name="c", subcore_axis_name="s")


def _wid():
    return lax.axis_index("s") * NC + lax.axis_index("c")


@functools.partial(
    pl.kernel,
    out_type=jax.ShapeDtypeStruct((NTILES, NP), jnp.float32),
    mesh=_mesh,
    scratch_types=[
        pltpu.VMEM((CK1,), jnp.int32),
        pltpu.VMEM((CK1,), jnp.float32),
        pltpu.VMEM((NP,), jnp.float32),
    ],
)
def _deg_partials(dst_hbm, ew_hbm, out_hbm, dstc, ewc, dacc):
    wid = _wid()
    z16 = jnp.zeros((16,), jnp.float32)

    def zbody(i, c):
        dacc[pl.ds(i * 16, 16)] = z16
        return c

    lax.fori_loop(0, NP // 16, zbody, 0)
    ebase = wid * ET1

    def chunk(c, carry):
        cb = ebase + c * CK1
        pltpu.sync_copy(dst_hbm.at[pl.ds(cb, CK1)], dstc)
        pltpu.sync_copy(ew_hbm.at[pl.ds(cb, CK1)], ewc)

        def vbody(i, cc):
            sl = pl.ds(i * 16, 16)
            plsc.addupdate_scatter(dacc, [dstc[sl]], ewc[sl])
            return cc

        lax.fori_loop(0, CK1 // 16, vbody, 0)
        return carry

    lax.fori_loop(0, ET1 // CK1, chunk, 0)
    pltpu.sync_copy(dacc, out_hbm.at[wid])


@functools.partial(
    pl.kernel,
    out_type=(
        jax.ShapeDtypeStruct((NP,), jnp.float32),
        jax.ShapeDtypeStruct((NP,), jnp.float32),
    ),
    mesh=_mesh,
    scratch_types=[
        pltpu.VMEM((STRIP,), jnp.float32),
        pltpu.VMEM((STRIP,), jnp.float32),
    ],
)
def _deg_reduce(degp_hbm, deg_hbm, rdeg_hbm, tmp, acc):
    wid = _wid()
    base = wid * STRIP
    z16 = jnp.zeros((16,), jnp.float32)

    def zbody(i, c):
        acc[pl.ds(i * 16, 16)] = z16
        return c

    lax.fori_loop(0, STRIP // 16, zbody, 0)

    def rbody(r, c):
        pltpu.sync_copy(degp_hbm.at[r, pl.ds(base, STRIP)], tmp)

        def abody(i, cc):
            sl = pl.ds(i * 16, 16)
            acc[sl] = acc[sl] + tmp[sl]
            return cc

        lax.fori_loop(0, STRIP // 16, abody, 0)
        return c

    lax.fori_loop(0, NTILES, rbody, 0)

    def nbody(i, c):
        sl = pl.ds(i * 16, 16)
        v = acc[sl] + EPS
        acc[sl] = v
        h = 0.5 * v
        bits = plsc.bitcast(v, jnp.int32)
        bits = 0x5F3759DF - (bits >> 1)
        y = plsc.bitcast(bits, jnp.float32)
        y = y * (1.5 - h * y * y)
        y = y * (1.5 - h * y * y)
        y = y * (1.5 - h * y * y)
        tmp[sl] = y
        return c

    lax.fori_loop(0, STRIP // 16, nbody, 0)
    pltpu.sync_copy(acc, deg_hbm.at[pl.ds(base, STRIP)])
    pltpu.sync_copy(tmp, rdeg_hbm.at[pl.ds(base, STRIP)])


@functools.partial(
    pl.kernel,
    out_type=(
        jax.ShapeDtypeStruct((NP * D,), jnp.float32),
        jax.ShapeDtypeStruct((NP * D,), jnp.float32),
        jax.ShapeDtypeStruct((NP * D,), jnp.float32),
        jax.ShapeDtypeStruct((NP,), jnp.float32),
    ),
    mesh=_mesh,
    scratch_types=[
        pltpu.VMEM((NP,), jnp.float32),
        pltpu.VMEM((C2,), jnp.int32),
        pltpu.VMEM((BUFCAP,), jnp.int32),
        pltpu.VMEM((BUFCAP,), jnp.int32),
        pltpu.VMEM((B,), jnp.int32),
        pltpu.VMEM((B,), jnp.int32),
        pltpu.VMEM((B,), jnp.float32),
        pltpu.VMEM((B,), jnp.float32),
        pltpu.VMEM((B, D), jnp.float32),
        pltpu.VMEM((RH * D,), jnp.float32),
        pltpu.VMEM((RH * D,), jnp.float32),
        pltpu.VMEM((RH * D,), jnp.float32),
        pltpu.VMEM((RH,), jnp.float32),
        pltpu.SMEM((B,), jnp.int32),
        pltpu.SMEM((B,), jnp.float32),
        pltpu.SMEM((B,), jnp.float32),
        pltpu.SemaphoreType.DMA,
        pltpu.SemaphoreType.DMA,
    ],
)
def _edge_aggregate(src_hbm, dst_hbm, ew_hbm, x_hbm, rdeg_hbm,
                    osum, ogcn, omax, osn,
                    rdegb, dstchunk, idbuf, dstbuf, idbatch, srcb, ewb,
                    normb, rows, accs, accg, accm, accn, smd, smw, smn,
                    sem1, sem2):
    wid = _wid()
    pltpu.sync_copy(rdeg_hbm, rdegb)
    iota = lax.iota(jnp.int32, 16)
    z16 = jnp.zeros((16,), jnp.float32)
    zi16 = jnp.zeros((16,), jnp.int32)
    neg16 = jnp.full((16,), -jnp.inf, jnp.float32)

    for p in range(NPASS):
        base = (p * NTILES + wid) * RH
        bse = jnp.full((16,), 1, jnp.int32) * base
        top = bse + RH

        def ibody(i, c):
            sl = pl.ds(i * 16, 16)
            accs[sl] = z16
            accg[sl] = z16
            accm[sl] = neg16
            return c

        lax.fori_loop(0, RH * D // 16, ibody, 0)

        def i2body(i, c):
            accn[pl.ds(i * 16, 16)] = z16
            return c

        lax.fori_loop(0, RH // 16, i2body, 0)

        def i3body(i, c):
            sl = pl.ds(i * 16, 16)
            idbuf[sl] = zi16
            dstbuf[sl] = bse
            return c

        lax.fori_loop(0, BUFCAP // 16, i3body, 0)

        def process_batch(off, cnt):
            for j in range(B // 16):
                idbatch[pl.ds(j * 16, 16)] = idbuf[pl.ds(off + j * 16, 16)]
            cp1 = pltpu.async_copy(src_hbm.at[idbatch], srcb, sem1)
            cp2 = pltpu.async_copy(ew_hbm.at[idbatch], ewb, sem2)
            cp1.wait()
            cp3 = pltpu.async_copy(x_hbm.at[srcb], rows, sem1)
            cp2.wait()
            for j in range(B // 16):
                sl = pl.ds(j * 16, 16)
                sv = srcb[sl]
                dv = dstbuf[pl.ds(off + j * 16, 16)]
                rs = plsc.load_gather(rdegb, [sv])
                rd = plsc.load_gather(rdegb, [dv])
                nv = ewb[sl] * rs * rd
                normb[sl] = nv
                m = (iota + j * 16) < cnt
                plsc.addupdate_scatter(accn, [dv - bse], nv, mask=m)
            pltpu.sync_copy(dstbuf.at[pl.ds(off, B)], smd)
            pltpu.sync_copy(ewb, smw)
            pltpu.sync_copy(normb, smn)
            cp3.wait()

            def ebody(e, c):
                dsc = smd[e]
                wsc = smw[e]
                nsc = smn[e]
                o = (dsc - base) * D
                for j in range(D // 16):
                    slr = pl.ds(e * D + j * 16, 16)
                    sla = pl.ds(o + j * 16, 16)
                    xv = rows[slr]
                    ms = xv * wsc
                    plsc.addupdate(accs.at[sla], ms)
                    plsc.addupdate(accg.at[sla], xv * nsc)
                    accm[sla] = jnp.maximum(accm[sla], ms)
                return c

            lax.fori_loop(0, cnt, ebody, 0)

        def chunk_body(c, wp):
            cb = c * C2
            pltpu.sync_copy(dst_hbm.at[pl.ds(cb, C2)], dstchunk)

            def vbody(i, w):
                dv = dstchunk[pl.ds(i * 16, 16)]
                own = (dv >= bse) & (dv < top)
                idv = iota + (cb + i * 16)
                plsc.store_compressed(idbuf.at[pl.ds(w, 16)], idv, mask=own)
                plsc.store_compressed(dstbuf.at[pl.ds(w, 16)], dv, mask=own)
                return w + jnp.max(plsc.all_reduce_population_count(own))

            wp = lax.fori_loop(0, C2 // 16, vbody, wp)
            nb = wp // B

            def bbody(k, cc):
                process_batch(k * B, B)
                return cc

            lax.fori_loop(0, nb, bbody, 0)
            rem = wp - nb * B
            for j in range(B // 16):
                sl = pl.ds(j * 16, 16)
                idbuf[sl] = idbuf[pl.ds(nb * B + j * 16, 16)]
                dstbuf[sl] = dstbuf[pl.ds(nb * B + j * 16, 16)]
            return rem

        wpf = lax.fori_loop(0, NCH2, chunk_body, jnp.int32(0))
        process_batch(0, wpf)

        pltpu.sync_copy(accs, osum.at[pl.ds(base * D, RH * D)])
        pltpu.sync_copy(accg, ogcn.at[pl.ds(base * D, RH * D)])
        pltpu.sync_copy(accm, omax.at[pl.ds(base * D, RH * D)])
        pltpu.sync_copy(accn, osn.at[pl.ds(base, RH)])


GB = 256


def _dense_body(wref, xr, asr, agr, amr, degr, snr,
                Wg, bg, Wss, Wsn, bs, W1, b1, W2, b2, Wme, bme, Wmx, bmx,
                outr):
    w0 = wref[0]
    w1 = wref[1]
    w2 = wref[2]
    w3 = wref[3]
    w4 = wref[4]
    xv = xr[...]
    s = asr[...]
    mean = s / degr[...]
    amax = amr[...]
    amax = jnp.where(jnp.isfinite(amax), amax, 0.0)
    f32 = jnp.float32
    h = jnp.maximum(
        jnp.dot(xv + s, W1[...], preferred_element_type=f32) + b1[...], 0.0)
    out = (jnp.dot(xv, Wss[...], preferred_element_type=f32) * w1
           + jnp.dot(mean, w1 * Wsn[...] + w3 * Wme[...],
                     preferred_element_type=f32)
           + jnp.dot(agr[...], Wg[...], preferred_element_type=f32) * w0
           + jnp.dot(amax, Wmx[...], preferred_element_type=f32) * w4
           + jnp.dot(h, W2[...], preferred_element_type=f32) * w2
           + (w1 * bs[...] + w2 * b2[...] + w3 * bme[...] + w4 * bmx[...])
           + (snr[...] * w0) * bg[...])
    outr[...] = out


def _dense_mix(weights, xp, aggs, aggg, aggm, deg2, sn2,
               W_gcn, b_gcn, W_sage_self, W_sage_neigh, b_sage,
               W_gin1, b_gin1, W_gin2, b_gin2, W_mean, b_mean, W_max, b_max):
    blk = lambda: pl.BlockSpec((GB, D), lambda i: (i, 0))
    col = lambda: pl.BlockSpec((GB, 1), lambda i: (i, 0))
    mat = lambda: pl.BlockSpec((D, D), lambda i: (0, 0))
    vec = lambda: pl.BlockSpec((1, D), lambda i: (0, 0))
    return pl.pallas_call(
        _dense_body,
        grid=(NP // GB,),
        in_specs=[
            pl.BlockSpec(memory_space=pltpu.SMEM),
            blk(), blk(), blk(), blk(), col(), col(),
            mat(), vec(), mat(), mat(), vec(), mat(), vec(), mat(), vec(),
            mat(), vec(), mat(), vec(),
        ],
        out_specs=pl.BlockSpec((GB, D), lambda i: (i, 0)),
        out_shape=jax.ShapeDtypeStruct((NP, D), jnp.float32),
    )(weights, xp, aggs, aggg, aggm, deg2, sn2,
      W_gcn, b_gcn.reshape(1, D), W_sage_self, W_sage_neigh,
      b_sage.reshape(1, D), W_gin1, b_gin1.reshape(1, D), W_gin2,
      b_gin2.reshape(1, D), W_mean, b_mean.reshape(1, D), W_max,
      b_max.reshape(1, D))


def kernel(x, weights, edge_index, edge_weights, with_linear,
           W_gcn, b_gcn, W_sage_self, W_sage_neigh, b_sage,
           W_gin1, b_gin1, W_gin2, b_gin2, W_mean, b_mean, W_max, b_max):
    src = edge_index[0]
    dst = edge_index[1]
    ew = edge_weights
    degp = _deg_partials(dst, ew)
    deg, rdeg = _deg_reduce(degp)
    aggs, aggg, aggm, snorm = _edge_aggregate(src, dst, ew, x, rdeg)
    xp = jnp.pad(x, ((0, NP - N), (0, 0)))
    out = _dense_mix(weights, xp,
                     aggs.reshape(NP, D), aggg.reshape(NP, D),
                     aggm.reshape(NP, D), deg.reshape(NP, 1),
                     snorm.reshape(NP, 1),
                     W_gcn, b_gcn, W_sage_self, W_sage_neigh, b_sage,
                     W_gin1, b_gin1, W_gin2, b_gin2, W_mean, b_mean,
                     W_max, b_max)
    return out[:N]


# speedup vs baseline: 4.4068x; 4.4068x over previous
import functools

import jax
import jax.numpy as jnp
from jax import lax
from jax.experimental import pallas as pl
from jax.experimental.pallas import tpu as pltpu
from jax.experimental.pallas import tpu_sc as plsc

N = 10000
E = 320000
D = 128
NC = 2
NS = 16
NTILES = NC * NS
NP = 10240
RH = 160
NPASS = 2
ET1 = E // NTILES
CK1 = 2000
C2 = 2560
NCH2 = E // C2
B = 128
BUFCAP = 2816
STRIP = NP // NTILES
EPS = 1e-6

def _wid():
    return lax.axis_index("s") * NC + lax.axis_index("c")


def _deg_partials(dst_hbm, ew_hbm, out_hbm, dstc, ewc, dacc):
    wid = _wid()
    z16 = jnp.zeros((16,), jnp.float32)

    def zbody(i, c):
        dacc[pl.ds(i * 16, 16)] = z16
        return c

    lax.fori_loop(0, NP // 16, zbody, 0)
    ebase = wid * ET1

    def chunk(c, carry):
        cb = ebase + c * CK1
        pltpu.sync_copy(dst_hbm.at[pl.ds(cb, CK1)], dstc)
        pltpu.sync_copy(ew_hbm.at[pl.ds(cb, CK1)], ewc)

        def vbody(i, cc):
            sl = pl.ds(i * 16, 16)
            plsc.addupdate_scatter(dacc, [dstc[sl]], ewc[sl])
            return cc

        lax.fori_loop(0, CK1 // 16, vbody, 0)
        return carry

    lax.fori_loop(0, ET1 // CK1, chunk, 0)
    pltpu.sync_copy(dacc, out_hbm.at[pl.ds(wid * NP, NP)])


def _deg_reduce(degp_hbm, deg_hbm, rdeg_hbm, tmp, acc):
    wid = _wid()
    base = wid * STRIP
    z16 = jnp.zeros((16,), jnp.float32)

    def zbody(i, c):
        acc[pl.ds(i * 16, 16)] = z16
        return c

    lax.fori_loop(0, STRIP // 16, zbody, 0)

    def rbody(r, c):
        pltpu.sync_copy(degp_hbm.at[pl.ds(r * NP + base, STRIP)], tmp)

        def abody(i, cc):
            sl = pl.ds(i * 16, 16)
            acc[sl] = acc[sl] + tmp[sl]
            return cc

        lax.fori_loop(0, STRIP // 16, abody, 0)
        return c

    lax.fori_loop(0, NTILES, rbody, 0)

    def nbody(i, c):
        sl = pl.ds(i * 16, 16)
        v = acc[sl] + EPS
        acc[sl] = v
        h = 0.5 * v
        bits = plsc.bitcast(v, jnp.int32)
        bits = 0x5F3759DF - (bits >> 1)
        y = plsc.bitcast(bits, jnp.float32)
        y = y * (1.5 - h * y * y)
        y = y * (1.5 - h * y * y)
        y = y * (1.5 - h * y * y)
        tmp[sl] = y
        return c

    lax.fori_loop(0, STRIP // 16, nbody, 0)
    pltpu.sync_copy(acc, deg_hbm.at[pl.ds(base, STRIP)])
    pltpu.sync_copy(tmp, rdeg_hbm.at[pl.ds(base, STRIP)])


def _edge_aggregate(src_hbm, dst_hbm, ew_hbm, x_hbm, rdeg_hbm,
                    osum, ogcn, omax, osn,
                    rdegb, dstchunk, idbuf, dstbuf, idbatch, srcb, ewb,
                    normb, rows, accs, accg, accm, accn,
                    sem1, sem2):
    wid = _wid()
    pltpu.sync_copy(rdeg_hbm, rdegb)
    iota = lax.iota(jnp.int32, 16)
    z16 = jnp.zeros((16,), jnp.float32)
    zi16 = jnp.zeros((16,), jnp.int32)
    neg16 = jnp.full((16,), -jnp.inf, jnp.float32)

    for p in range(NPASS):
        base = (p * NTILES + wid) * RH
        bse = jnp.full((16,), 1, jnp.int32) * base
        top = bse + RH

        def ibody(i, c):
            sl = pl.ds(i * 16, 16)
            accs[sl] = z16
            accg[sl] = z16
            accm[sl] = neg16
            return c

        lax.fori_loop(0, RH * D // 16, ibody, 0)

        def i2body(i, c):
            accn[pl.ds(i * 16, 16)] = z16
            return c

        lax.fori_loop(0, RH // 16, i2body, 0)

        def i3body(i, c):
            sl = pl.ds(i * 16, 16)
            idbuf[sl] = zi16
            dstbuf[sl] = bse
            return c

        lax.fori_loop(0, BUFCAP // 16, i3body, 0)

        def process_batch(off, cnt):
            for j in range(B // 16):
                idbatch[pl.ds(j * 16, 16)] = idbuf[pl.ds(off + j * 16, 16)]
            cp1 = pltpu.async_copy(src_hbm.at[idbatch], srcb, sem1)
            cp2 = pltpu.async_copy(ew_hbm.at[idbatch], ewb.at[pl.ds(0, B)],
                                   sem2)
            cp1.wait()
            cp3 = pltpu.async_copy(x_hbm.at[srcb], rows, sem1)
            cp2.wait()
            for j in range(B // 16):
                sl = pl.ds(j * 16, 16)
                sv = srcb[sl]
                dv = dstbuf[pl.ds(off + j * 16, 16)]
                rs = plsc.load_gather(rdegb, [sv])
                rd = plsc.load_gather(rdegb, [dv])
                nv = ewb[sl] * rs * rd
                normb[sl] = nv
                m = (iota + j * 16) < cnt
                plsc.addupdate_scatter(accn, [dv - bse], nv, mask=m)
            cp3.wait()

            def ebody(e, c):
                dsc = dstbuf[pl.ds(off + e, 16)][0]
                wsc = ewb[pl.ds(e, 16)][0]
                nsc = normb[pl.ds(e, 16)][0]
                o = (dsc - base) * D
                for j in range(D // 16):
                    sla = pl.ds(o + j * 16, 16)
                    xv = rows[e, pl.ds(j * 16, 16)]
                    ms = xv * wsc
                    plsc.addupdate(accs.at[sla], ms)
                    plsc.addupdate(accg.at[sla], xv * nsc)
                    accm[sla] = jnp.maximum(accm[sla], ms)
                return c

            lax.fori_loop(0, cnt, ebody, 0)

        def chunk_body(c, wp):
            cb = c * C2
            pltpu.sync_copy(dst_hbm.at[pl.ds(cb, C2)], dstchunk)

            def vbody(i, w):
                dv = dstchunk[pl.ds(i * 16, 16)]
                own = (dv >= bse) & (dv < top)
                idv = iota + (cb + i * 16)
                plsc.store_compressed(idbuf.at[pl.ds(w, 16)], idv, mask=own)
                plsc.store_compressed(dstbuf.at[pl.ds(w, 16)], dv, mask=own)
                return w + jnp.max(plsc.all_reduce_population_count(own))

            wp = lax.fori_loop(0, C2 // 16, vbody, wp)
            nb = wp // B

            def bbody(k, cc):
                process_batch(k * B, B)
                return cc

            lax.fori_loop(0, nb, bbody, 0)
            rem = wp - nb * B
            for j in range(B // 16):
                sl = pl.ds(j * 16, 16)
                idbuf[sl] = idbuf[pl.ds(nb * B + j * 16, 16)]
                dstbuf[sl] = dstbuf[pl.ds(nb * B + j * 16, 16)]
            return rem

        wpf = lax.fori_loop(0, NCH2, chunk_body, jnp.int32(0))
        process_batch(0, wpf)

        pltpu.sync_copy(accs, osum.at[pl.ds(base * D, RH * D)])
        pltpu.sync_copy(accg, ogcn.at[pl.ds(base * D, RH * D)])
        pltpu.sync_copy(accm, omax.at[pl.ds(base * D, RH * D)])
        pltpu.sync_copy(accn, osn.at[pl.ds(base, RH)])


GB = 256


def _dense_body(wref, xr, asr, agr, amr, degr, snr,
                Wg, bg, Wss, Wsn, bs, W1, b1, W2, b2, Wme, bme, Wmx, bmx,
                outr):
    w0 = wref[0]
    w1 = wref[1]
    w2 = wref[2]
    w3 = wref[3]
    w4 = wref[4]
    xv = xr[...]
    s = asr[...]
    mean = s / degr[...]
    amax = amr[...]
    amax = jnp.where(jnp.isfinite(amax), amax, 0.0)
    f32 = jnp.float32
    h = jnp.maximum(
        jnp.dot(xv + s, W1[...], preferred_element_type=f32) + b1[...], 0.0)
    out = (jnp.dot(xv, Wss[...], preferred_element_type=f32) * w1
           + jnp.dot(mean, w1 * Wsn[...] + w3 * Wme[...],
                     preferred_element_type=f32)
           + jnp.dot(agr[...], Wg[...], preferred_element_type=f32) * w0
           + jnp.dot(amax, Wmx[...], preferred_element_type=f32) * w4
           + jnp.dot(h, W2[...], preferred_element_type=f32) * w2
           + (w1 * bs[...] + w2 * b2[...] + w3 * bme[...] + w4 * bmx[...])
           + (snr[...] * w0) * bg[...])
    outr[...] = out


def _dense_mix(weights, xp, aggs, aggg, aggm, deg2, sn2,
               W_gcn, b_gcn, W_sage_self, W_sage_neigh, b_sage,
               W_gin1, b_gin1, W_gin2, b_gin2, W_mean, b_mean, W_max, b_max):
    blk = lambda: pl.BlockSpec((GB, D), lambda i: (i, 0))
    col = lambda: pl.BlockSpec((GB, 1), lambda i: (i, 0))
    mat = lambda: pl.BlockSpec((D, D), lambda i: (0, 0))
    vec = lambda: pl.BlockSpec((1, D), lambda i: (0, 0))
    return pl.pallas_call(
        _dense_body,
        grid=(NP // GB,),
        in_specs=[
            pl.BlockSpec(memory_space=pltpu.SMEM),
            blk(), blk(), blk(), blk(), col(), col(),
            mat(), vec(), mat(), mat(), vec(), mat(), vec(), mat(), vec(),
            mat(), vec(), mat(), vec(),
        ],
        out_specs=pl.BlockSpec((GB, D), lambda i: (i, 0)),
        out_shape=jax.ShapeDtypeStruct((NP, D), jnp.float32),
    )(weights, xp, aggs, aggg, aggm, deg2, sn2,
      W_gcn, b_gcn.reshape(1, D), W_sage_self, W_sage_neigh,
      b_sage.reshape(1, D), W_gin1, b_gin1.reshape(1, D), W_gin2,
      b_gin2.reshape(1, D), W_mean, b_mean.reshape(1, D), W_max,
      b_max.reshape(1, D))


@functools.lru_cache(maxsize=1)
def _sc_kernels():
    mesh = plsc.VectorSubcoreMesh(core_axis_name="c", subcore_axis_name="s",
                                  num_cores=NC, num_subcores=NS)
    f32 = jnp.float32
    deg_partials = pl.kernel(
        _deg_partials,
        out_type=jax.ShapeDtypeStruct((NTILES * NP,), f32),
        mesh=mesh,
        compiler_params=pltpu.CompilerParams(needs_layout_passes=False),
        scratch_types=[
            pltpu.VMEM((CK1,), jnp.int32),
            pltpu.VMEM((CK1,), f32),
            pltpu.VMEM((NP,), f32),
        ],
    )
    deg_reduce = pl.kernel(
        _deg_reduce,
        out_type=(
            jax.ShapeDtypeStruct((NP,), f32),
            jax.ShapeDtypeStruct((NP,), f32),
        ),
        mesh=mesh,
        compiler_params=pltpu.CompilerParams(needs_layout_passes=False),
        scratch_types=[
            pltpu.VMEM((STRIP,), f32),
            pltpu.VMEM((STRIP,), f32),
        ],
    )
    edge_aggregate = pl.kernel(
        _edge_aggregate,
        out_type=(
            jax.ShapeDtypeStruct((NP * D,), f32),
            jax.ShapeDtypeStruct((NP * D,), f32),
            jax.ShapeDtypeStruct((NP * D,), f32),
            jax.ShapeDtypeStruct((NP,), f32),
        ),
        mesh=mesh,
        compiler_params=pltpu.CompilerParams(needs_layout_passes=False),
        scratch_types=[
            pltpu.VMEM((NP,), f32),
            pltpu.VMEM((C2,), jnp.int32),
            pltpu.VMEM((BUFCAP,), jnp.int32),
            pltpu.VMEM((BUFCAP,), jnp.int32),
            pltpu.VMEM((B,), jnp.int32),
            pltpu.VMEM((B,), jnp.int32),
            pltpu.VMEM((B + 16,), f32),
            pltpu.VMEM((B + 16,), f32),
            pltpu.VMEM((B, D), f32),
            pltpu.VMEM((RH * D,), f32),
            pltpu.VMEM((RH * D,), f32),
            pltpu.VMEM((RH * D,), f32),
            pltpu.VMEM((RH,), f32),
            pltpu.SemaphoreType.DMA,
            pltpu.SemaphoreType.DMA,
        ],
    )
    return deg_partials, deg_reduce, edge_aggregate


def kernel(x, weights, edge_index, edge_weights, with_linear,
           W_gcn, b_gcn, W_sage_self, W_sage_neigh, b_sage,
           W_gin1, b_gin1, W_gin2, b_gin2, W_mean, b_mean, W_max, b_max):
    src = edge_index[0]
    dst = edge_index[1]
    ew = edge_weights
    deg_partials, deg_reduce, edge_aggregate = _sc_kernels()
    degp = deg_partials(dst, ew)
    deg, rdeg = deg_reduce(degp)
    aggs, aggg, aggm, snorm = edge_aggregate(src, dst, ew, x, rdeg)
    xp = jnp.pad(x, ((0, NP - N), (0, 0)))
    out = _dense_mix(weights, xp,
                     aggs.reshape(NP, D), aggg.reshape(NP, D),
                     aggm.reshape(NP, D), deg.reshape(NP, 1),
                     snorm.reshape(NP, 1),
                     W_gcn, b_gcn, W_sage_self, W_sage_neigh, b_sage,
                     W_gin1, b_gin1, W_gin2, b_gin2, W_mean, b_mean,
                     W_max, b_max)
    return out[:N]


# speedup vs baseline: 4.8581x; 1.1024x over previous
import functools

import jax
import jax.numpy as jnp
from jax import lax
from jax.experimental import pallas as pl
from jax.experimental.pallas import tpu as pltpu
from jax.experimental.pallas import tpu_sc as plsc

N = 10000
E = 320000
D = 128
NC = 2
NS = 16
NTILES = NC * NS
NP = 10240
RH = 160
NPASS = 2
ET1 = E // NTILES
CK1 = 2000
C2 = 2000
NCH2 = E // C2
B = 128
BUFCAP = 2176
STRIP = NP // NTILES
EPS = 1e-6

def _wid():
    return lax.axis_index("s") * NC + lax.axis_index("c")


def _deg_partials(dst_hbm, ew_hbm, out_hbm, dstc, ewc, dacc):
    wid = _wid()
    z16 = jnp.zeros((16,), jnp.float32)

    def zbody(i, c):
        dacc[pl.ds(i * 16, 16)] = z16
        return c

    lax.fori_loop(0, NP // 16, zbody, 0)
    ebase = wid * ET1

    def chunk(c, carry):
        cb = ebase + c * CK1
        pltpu.sync_copy(dst_hbm.at[pl.ds(cb, CK1)], dstc)
        pltpu.sync_copy(ew_hbm.at[pl.ds(cb, CK1)], ewc)

        def vbody(i, cc):
            sl = pl.ds(i * 16, 16)
            plsc.addupdate_scatter(dacc, [dstc[sl]], ewc[sl])
            return cc

        lax.fori_loop(0, CK1 // 16, vbody, 0)
        return carry

    lax.fori_loop(0, ET1 // CK1, chunk, 0)
    pltpu.sync_copy(dacc, out_hbm.at[pl.ds(wid * NP, NP)])


def _deg_reduce(degp_hbm, deg_hbm, rdeg_hbm, tmp, acc):
    wid = _wid()
    base = wid * STRIP
    z16 = jnp.zeros((16,), jnp.float32)

    def zbody(i, c):
        acc[pl.ds(i * 16, 16)] = z16
        return c

    lax.fori_loop(0, STRIP // 16, zbody, 0)

    def rbody(r, c):
        pltpu.sync_copy(degp_hbm.at[pl.ds(r * NP + base, STRIP)], tmp)

        def abody(i, cc):
            sl = pl.ds(i * 16, 16)
            acc[sl] = acc[sl] + tmp[sl]
            return cc

        lax.fori_loop(0, STRIP // 16, abody, 0)
        return c

    lax.fori_loop(0, NTILES, rbody, 0)

    def nbody(i, c):
        sl = pl.ds(i * 16, 16)
        v = acc[sl] + EPS
        acc[sl] = v
        h = 0.5 * v
        bits = plsc.bitcast(v, jnp.int32)
        bits = 0x5F3759DF - (bits >> 1)
        y = plsc.bitcast(bits, jnp.float32)
        y = y * (1.5 - h * y * y)
        y = y * (1.5 - h * y * y)
        y = y * (1.5 - h * y * y)
        tmp[sl] = y
        return c

    lax.fori_loop(0, STRIP // 16, nbody, 0)
    pltpu.sync_copy(acc, deg_hbm.at[pl.ds(base, STRIP)])
    pltpu.sync_copy(tmp, rdeg_hbm.at[pl.ds(base, STRIP)])


def _edge_aggregate(src_hbm, dst_hbm, ew_hbm, x_hbm, rdeg_hbm,
                    osum, ogcn, omax, osn,
                    rdegb, dstc0, dstc1, idbuf, dstbuf, idbatch, srcb, ewb,
                    normb, rows, accs, accg, accm, accn,
                    sem1, sem2, semd0, semd1):
    wid = _wid()
    pltpu.sync_copy(rdeg_hbm, rdegb)
    iota = lax.iota(jnp.int32, 16)
    z16 = jnp.zeros((16,), jnp.float32)
    zi16 = jnp.zeros((16,), jnp.int32)
    neg16 = jnp.full((16,), -jnp.inf, jnp.float32)

    for p in range(NPASS):
        base = (p * NTILES + wid) * RH
        bse = jnp.full((16,), 1, jnp.int32) * base
        top = bse + RH

        def ibody(i, c):
            sl = pl.ds(i * 16, 16)
            accs[sl] = z16
            accg[sl] = z16
            accm[sl] = neg16
            return c

        lax.fori_loop(0, RH * D // 16, ibody, 0)

        def i2body(i, c):
            accn[pl.ds(i * 16, 16)] = z16
            return c

        lax.fori_loop(0, RH // 16, i2body, 0)

        def i3body(i, c):
            sl = pl.ds(i * 16, 16)
            idbuf[sl] = zi16
            dstbuf[sl] = bse
            return c

        lax.fori_loop(0, BUFCAP // 16, i3body, 0)

        def process_batch(off, cnt):
            for j in range(B // 16):
                idbatch[pl.ds(j * 16, 16)] = idbuf[pl.ds(off + j * 16, 16)]
            cp1 = pltpu.async_copy(src_hbm.at[idbatch], srcb, sem1)
            cp2 = pltpu.async_copy(ew_hbm.at[idbatch], ewb.at[pl.ds(0, B)],
                                   sem2)
            cp1.wait()
            cp3 = pltpu.async_copy(x_hbm.at[srcb], rows, sem1)
            cp2.wait()
            for j in range(B // 16):
                sl = pl.ds(j * 16, 16)
                sv = srcb[sl]
                dv = dstbuf[pl.ds(off + j * 16, 16)]
                rs = plsc.load_gather(rdegb, [sv])
                rd = plsc.load_gather(rdegb, [dv])
                nv = ewb[sl] * rs * rd
                normb[sl] = nv
                m = (iota + j * 16) < cnt
                plsc.addupdate_scatter(accn, [dv - bse], nv, mask=m)
            cp3.wait()

            def ebody(e, c):
                dsc = dstbuf[pl.ds(off + e, 16)][0]
                wsc = ewb[pl.ds(e, 16)][0]
                nsc = normb[pl.ds(e, 16)][0]
                o = (dsc - base) * D
                for j in range(D // 16):
                    sla = pl.ds(o + j * 16, 16)
                    xv = rows[e, pl.ds(j * 16, 16)]
                    ms = xv * wsc
                    plsc.addupdate(accs.at[sla], ms)
                    plsc.addupdate(accg.at[sla], xv * nsc)
                    accm[sla] = jnp.maximum(accm[sla], ms)
                return c

            lax.fori_loop(0, cnt, ebody, 0)

        def scan_chunk(dref, cb, wp):
            def vbody(i, w):
                dv = dref[pl.ds(i * 16, 16)]
                own = (dv >= bse) & (dv < top)
                idv = iota + (cb + i * 16)
                plsc.store_compressed(idbuf.at[pl.ds(w, 16)], idv, mask=own)
                plsc.store_compressed(dstbuf.at[pl.ds(w, 16)], dv, mask=own)
                return w + jnp.max(plsc.all_reduce_population_count(own))

            wp = lax.fori_loop(0, C2 // 16, vbody, wp)
            nb = wp // B

            def bbody(k, cc):
                process_batch(k * B, B)
                return cc

            lax.fori_loop(0, nb, bbody, 0)
            rem = wp - nb * B
            for j in range(B // 16):
                sl = pl.ds(j * 16, 16)
                idbuf[sl] = idbuf[pl.ds(nb * B + j * 16, 16)]
                dstbuf[sl] = dstbuf[pl.ds(nb * B + j * 16, 16)]
            return rem

        pltpu.async_copy(dst_hbm.at[pl.ds(0, C2)], dstc0, semd0)
        pltpu.async_copy(dst_hbm.at[pl.ds(C2, C2)], dstc1, semd1)

        def pair_body(g, wp):
            for b, (dref, sd) in enumerate(((dstc0, semd0), (dstc1, semd1))):
                c = g * 2 + b
                pltpu.make_async_copy(dst_hbm.at[pl.ds(0, C2)],
                                      dref, sd).wait()
                wp = scan_chunk(dref, c * C2, wp)
                nxt = jnp.minimum((c + 2) * C2, E - C2)
                pltpu.async_copy(dst_hbm.at[pl.ds(nxt, C2)], dref, sd)
            return wp

        wpf = lax.fori_loop(0, NCH2 // 2, pair_body, jnp.int32(0))
        pltpu.make_async_copy(dst_hbm.at[pl.ds(0, C2)], dstc0, semd0).wait()
        pltpu.make_async_copy(dst_hbm.at[pl.ds(0, C2)], dstc1, semd1).wait()
        process_batch(0, wpf)

        pltpu.sync_copy(accs, osum.at[pl.ds(base * D, RH * D)])
        pltpu.sync_copy(accg, ogcn.at[pl.ds(base * D, RH * D)])
        pltpu.sync_copy(accm, omax.at[pl.ds(base * D, RH * D)])
        pltpu.sync_copy(accn, osn.at[pl.ds(base, RH)])


GB = 256


def _dense_body(wref, xr, asr, agr, amr, degr, snr,
                Wg, bg, Wss, Wsn, bs, W1, b1, W2, b2, Wme, bme, Wmx, bmx,
                outr):
    w0 = wref[0]
    w1 = wref[1]
    w2 = wref[2]
    w3 = wref[3]
    w4 = wref[4]
    xv = xr[...]
    s = asr[...]
    mean = s / degr[...]
    amax = amr[...]
    amax = jnp.where(jnp.isfinite(amax), amax, 0.0)
    f32 = jnp.float32
    h = jnp.maximum(
        jnp.dot(xv + s, W1[...], preferred_element_type=f32) + b1[...], 0.0)
    out = (jnp.dot(xv, Wss[...], preferred_element_type=f32) * w1
           + jnp.dot(mean, w1 * Wsn[...] + w3 * Wme[...],
                     preferred_element_type=f32)
           + jnp.dot(agr[...], Wg[...], preferred_element_type=f32) * w0
           + jnp.dot(amax, Wmx[...], preferred_element_type=f32) * w4
           + jnp.dot(h, W2[...], preferred_element_type=f32) * w2
           + (w1 * bs[...] + w2 * b2[...] + w3 * bme[...] + w4 * bmx[...])
           + (snr[...] * w0) * bg[...])
    outr[...] = out


def _dense_mix(weights, xp, aggs, aggg, aggm, deg2, sn2,
               W_gcn, b_gcn, W_sage_self, W_sage_neigh, b_sage,
               W_gin1, b_gin1, W_gin2, b_gin2, W_mean, b_mean, W_max, b_max):
    blk = lambda: pl.BlockSpec((GB, D), lambda i: (i, 0))
    col = lambda: pl.BlockSpec((GB, 1), lambda i: (i, 0))
    mat = lambda: pl.BlockSpec((D, D), lambda i: (0, 0))
    vec = lambda: pl.BlockSpec((1, D), lambda i: (0, 0))
    return pl.pallas_call(
        _dense_body,
        grid=(NP // GB,),
        in_specs=[
            pl.BlockSpec(memory_space=pltpu.SMEM),
            blk(), blk(), blk(), blk(), col(), col(),
            mat(), vec(), mat(), mat(), vec(), mat(), vec(), mat(), vec(),
            mat(), vec(), mat(), vec(),
        ],
        out_specs=pl.BlockSpec((GB, D), lambda i: (i, 0)),
        out_shape=jax.ShapeDtypeStruct((NP, D), jnp.float32),
    )(weights, xp, aggs, aggg, aggm, deg2, sn2,
      W_gcn, b_gcn.reshape(1, D), W_sage_self, W_sage_neigh,
      b_sage.reshape(1, D), W_gin1, b_gin1.reshape(1, D), W_gin2,
      b_gin2.reshape(1, D), W_mean, b_mean.reshape(1, D), W_max,
      b_max.reshape(1, D))


@functools.lru_cache(maxsize=1)
def _sc_kernels():
    mesh = plsc.VectorSubcoreMesh(core_axis_name="c", subcore_axis_name="s",
                                  num_cores=NC, num_subcores=NS)
    f32 = jnp.float32
    deg_partials = pl.kernel(
        _deg_partials,
        out_type=jax.ShapeDtypeStruct((NTILES * NP,), f32),
        mesh=mesh,
        compiler_params=pltpu.CompilerParams(needs_layout_passes=False),
        scratch_types=[
            pltpu.VMEM((CK1,), jnp.int32),
            pltpu.VMEM((CK1,), f32),
            pltpu.VMEM((NP,), f32),
        ],
    )
    deg_reduce = pl.kernel(
        _deg_reduce,
        out_type=(
            jax.ShapeDtypeStruct((NP,), f32),
            jax.ShapeDtypeStruct((NP,), f32),
        ),
        mesh=mesh,
        compiler_params=pltpu.CompilerParams(needs_layout_passes=False),
        scratch_types=[
            pltpu.VMEM((STRIP,), f32),
            pltpu.VMEM((STRIP,), f32),
        ],
    )
    edge_aggregate = pl.kernel(
        _edge_aggregate,
        out_type=(
            jax.ShapeDtypeStruct((NP * D,), f32),
            jax.ShapeDtypeStruct((NP * D,), f32),
            jax.ShapeDtypeStruct((NP * D,), f32),
            jax.ShapeDtypeStruct((NP,), f32),
        ),
        mesh=mesh,
        compiler_params=pltpu.CompilerParams(needs_layout_passes=False),
        scratch_types=[
            pltpu.VMEM((NP,), f32),
            pltpu.VMEM((C2,), jnp.int32),
            pltpu.VMEM((C2,), jnp.int32),
            pltpu.VMEM((BUFCAP,), jnp.int32),
            pltpu.VMEM((BUFCAP,), jnp.int32),
            pltpu.VMEM((B,), jnp.int32),
            pltpu.VMEM((B,), jnp.int32),
            pltpu.VMEM((B + 16,), f32),
            pltpu.VMEM((B + 16,), f32),
            pltpu.VMEM((B, D), f32),
            pltpu.VMEM((RH * D,), f32),
            pltpu.VMEM((RH * D,), f32),
            pltpu.VMEM((RH * D,), f32),
            pltpu.VMEM((RH,), f32),
            pltpu.SemaphoreType.DMA,
            pltpu.SemaphoreType.DMA,
            pltpu.SemaphoreType.DMA,
            pltpu.SemaphoreType.DMA,
        ],
    )
    return deg_partials, deg_reduce, edge_aggregate


def kernel(x, weights, edge_index, edge_weights, with_linear,
           W_gcn, b_gcn, W_sage_self, W_sage_neigh, b_sage,
           W_gin1, b_gin1, W_gin2, b_gin2, W_mean, b_mean, W_max, b_max):
    src = edge_index[0]
    dst = edge_index[1]
    ew = edge_weights
    deg_partials, deg_reduce, edge_aggregate = _sc_kernels()
    degp = deg_partials(dst, ew)
    deg, rdeg = deg_reduce(degp)
    aggs, aggg, aggm, snorm = edge_aggregate(src, dst, ew, x, rdeg)
    xp = jnp.pad(x, ((0, NP - N), (0, 0)))
    out = _dense_mix(weights, xp,
                     aggs.reshape(NP, D), aggg.reshape(NP, D),
                     aggm.reshape(NP, D), deg.reshape(NP, 1),
                     snorm.reshape(NP, 1),
                     W_gcn, b_gcn, W_sage_self, W_sage_neigh, b_sage,
                     W_gin1, b_gin1, W_gin2, b_gin2, W_mean, b_mean,
                     W_max, b_max)
    return out[:N]
